# Initial kernel scaffold; baseline (speedup 1.0000x reference)
import functools

import jax
import jax.numpy as jnp
from jax import lax
from jax.experimental import pallas as pl
from jax.experimental.pallas import tpu as pltpu
from jax.experimental.pallas import tpu_sc as plsc

D = 128
NE = 10000
NR = 256
E = 320000
NC = 2
NS = 16
NW = NC * NS
CH = 128
CPT = 79
EPT = CH * CPT
EPAD = NW * EPT
ACC = 10016
PAD_DST = 10008
RPT = ACC // NS
OPT = NE // NS


@functools.partial(
    pl.kernel,
    mesh=plsc.VectorSubcoreMesh(core_axis_name="c", subcore_axis_name="s"),
    out_type=(
        jax.ShapeDtypeStruct((NC, NE, D), jnp.float32),
        jax.ShapeDtypeStruct((NW, ACC), jnp.float32),
    ),
    scratch_types=[
        pltpu.VMEM((CPT, CH), jnp.int32),
        pltpu.VMEM((CPT, CH), jnp.int32),
        pltpu.VMEM((CPT, CH), jnp.int32),
        pltpu.VMEM((CH, D), jnp.float32),
        pltpu.VMEM((CH, D), jnp.float32),
        pltpu.VMEM((CH, D), jnp.float32),
        pltpu.VMEM((ACC,), jnp.float32),
        pltpu.VMEM_SHARED((ACC, D), jnp.float32),
        pltpu.VMEM_SHARED((NR, D), jnp.float32),
        pltpu.SemaphoreType.DMA,
        pltpu.SemaphoreType.DMA,
    ],
)
def _sc_aggregate(x_hbm, negr_hbm, src_hbm, dst_hbm, et_hbm,
                  sp_out, degp_out,
                  src_v, dst_v, et_v, xrows, nrows, zbuf, deg_v,
                  acc, negr_sh, sem_x, sem_r):
    cid = lax.axis_index("c")
    sid = lax.axis_index("s")
    wid = cid * NS + sid

    pltpu.sync_copy(src_hbm.at[wid], src_v)
    pltpu.sync_copy(dst_hbm.at[wid], dst_v)
    pltpu.sync_copy(et_hbm.at[wid], et_v)

    z16 = jnp.zeros((16,), jnp.float32)

    def zrow(i, carry):
        for k in range(D // 16):
            zbuf[i, pl.ds(k * 16, 16)] = z16
        return carry
    lax.fori_loop(0, CH, zrow, 0)

    def zdeg(i, carry):
        deg_v[pl.ds(i * 16, 16)] = z16
        return carry
    lax.fori_loop(0, ACC // 16, zdeg, 0)

    base = sid * RPT
    off = 0
    rem = RPT
    while rem > 0:
        n = min(rem, CH)
        pltpu.sync_copy(zbuf.at[pl.ds(0, n)], acc.at[pl.ds(base + off, n)])
        off += n
        rem -= n

    @pl.when(sid == 0)
    def _():
        pltpu.sync_copy(negr_hbm, negr_sh)

    plsc.subcore_barrier()

    ones16 = jnp.ones((16,), jnp.float32)

    def dbody(j, carry):
        for k in range(CH // 16):
            idx = dst_v[j, pl.ds(k * 16, 16)]
            plsc.addupdate_scatter(deg_v, [idx], ones16)
        return carry
    lax.fori_loop(0, CPT, dbody, 0)

    def ebody(j, carry):
        pltpu.async_copy(x_hbm.at[src_v.at[j]], xrows, sem_x).wait()
        pltpu.async_copy(negr_sh.at[et_v.at[j]], nrows, sem_r).wait()
        pltpu.sync_copy(xrows, acc.at[dst_v.at[j]], add=True)
        pltpu.sync_copy(nrows, acc.at[dst_v.at[j]], add=True)
        return carry
    lax.fori_loop(0, CPT, ebody, 0)

    plsc.subcore_barrier()

    ob = sid * OPT
    pltpu.sync_copy(acc.at[pl.ds(ob, OPT)], sp_out.at[cid, pl.ds(ob, OPT)])
    pltpu.sync_copy(deg_v, degp_out.at[wid])


def _ln(h, g, b, eps=1e-5):
    mu = jnp.mean(h, axis=-1, keepdims=True)
    var = jnp.mean((h - mu) ** 2, axis=-1, keepdims=True)
    return (h - mu) / jnp.sqrt(var + eps) * g + b


def _tc_rel_body(rel_ref, W1_ref, g1_ref, b1_ref, W2_ref, g2_ref, b2_ref,
                 r1_ref, n0_ref, n1_ref, r2_ref):
    rel = rel_ref[...]
    n0_ref[...] = -rel
    h1 = jnp.dot(rel, W1_ref[...], preferred_element_type=jnp.float32)
    r1 = jnp.tanh(_ln(h1, g1_ref[...], b1_ref[...]))
    r1_ref[...] = r1
    n1_ref[...] = -r1
    h2 = jnp.dot(r1, W2_ref[...], preferred_element_type=jnp.float32)
    r2_ref[...] = jnp.tanh(_ln(h2, g2_ref[...], b2_ref[...]))


def _tc_rel(rel_emb, Wrel1, g_rel1, b_rel1, Wrel2, g_rel2, b_rel2):
    return pl.pallas_call(
        _tc_rel_body,
        out_shape=[jax.ShapeDtypeStruct((NR, D), jnp.float32)] * 4,
    )(rel_emb, Wrel1, g_rel1.reshape(1, D), b_rel1.reshape(1, D),
      Wrel2, g_rel2.reshape(1, D), b_rel2.reshape(1, D))


XB = 1250


def _tc_x_body(sp_ref, degp_ref, time_ref, Wt_ref, bt_ref, gt_ref, btl_ref,
               W_ref, g_ref, b_ref, out_ref):
    t = jnp.dot(time_ref[...], Wt_ref[...],
                preferred_element_type=jnp.float32) + bt_ref[...]
    t = _ln(t, gt_ref[...], btl_ref[...])
    S = sp_ref[0] + sp_ref[1]
    deg = jnp.sum(degp_ref[...], axis=0)
    degc = jnp.maximum(deg, 1.0)
    agg = (S + deg[:, None] * t) / degc[:, None]
    h = jnp.dot(agg, W_ref[...], preferred_element_type=jnp.float32)
    out_ref[...] = jnp.tanh(_ln(h, g_ref[...], b_ref[...]))


def _tc_x(sp, degp, time_emd, W_time, b_time, g_tln, b_tln, W, g, b):
    return pl.pallas_call(
        _tc_x_body,
        grid=(NE // XB,),
        in_specs=[
            pl.BlockSpec((NC, XB, D), lambda i: (0, i, 0)),
            pl.BlockSpec((NW, XB), lambda i: (0, i)),
            pl.BlockSpec((1, D), lambda i: (0, 0)),
            pl.BlockSpec((D, D), lambda i: (0, 0)),
            pl.BlockSpec((1, D), lambda i: (0, 0)),
            pl.BlockSpec((1, D), lambda i: (0, 0)),
            pl.BlockSpec((1, D), lambda i: (0, 0)),
            pl.BlockSpec((D, D), lambda i: (0, 0)),
            pl.BlockSpec((1, D), lambda i: (0, 0)),
            pl.BlockSpec((1, D), lambda i: (0, 0)),
        ],
        out_specs=pl.BlockSpec((XB, D), lambda i: (i, 0)),
        out_shape=jax.ShapeDtypeStruct((NE, D), jnp.float32),
    )(sp, degp, time_emd.reshape(1, D), W_time, b_time.reshape(1, D),
      g_tln.reshape(1, D), b_tln.reshape(1, D), W, g.reshape(1, D),
      b.reshape(1, D))


def kernel(ent_emb, rel_emb, time_emd, edge_index, edge_type,
           W_time, b_time, g_time_ln, b_time_ln,
           W1, Wrel1, g_ent1, b_ent1, g_rel1, b_rel1,
           W2, Wrel2, g_ent2, b_ent2, g_rel2, b_rel2):
    src = edge_index[0].astype(jnp.int32)
    dst = edge_index[1].astype(jnp.int32)
    et = edge_type.astype(jnp.int32)
    pad = EPAD - E
    srcp = jnp.pad(src, (0, pad)).reshape(NW, CPT, CH)
    dstp = jnp.pad(dst, (0, pad), constant_values=PAD_DST).reshape(NW, CPT, CH)
    etp = jnp.pad(et, (0, pad)).reshape(NW, CPT, CH)

    r1, negr0, negr1, r2 = _tc_rel(rel_emb, Wrel1, g_rel1, b_rel1,
                                   Wrel2, g_rel2, b_rel2)

    sp1, degp = _sc_aggregate(ent_emb, negr0, srcp, dstp, etp)
    x1 = _tc_x(sp1, degp, time_emd, W_time, b_time, g_time_ln, b_time_ln,
               W1, g_ent1, b_ent1)
    sp2, _ = _sc_aggregate(x1, negr1, srcp, dstp, etp)
    x2 = _tc_x(sp2, degp, time_emd, W_time, b_time, g_time_ln, b_time_ln,
               W2, g_ent2, b_ent2)
    return (x2, r2)


# speedup vs baseline: 3.7310x; 3.7310x over previous
import functools

import jax
import jax.numpy as jnp
from jax import lax
from jax.experimental import pallas as pl
from jax.experimental.pallas import tpu as pltpu
from jax.experimental.pallas import tpu_sc as plsc

D = 128
HD = D // 2
NE = 10000
NR = 256
E = 320000
NC = 2
NS = 16
CH = 128
CPT = 158
EPT = CH * CPT
EPAD = NS * EPT
ACC = 10112
PAD_DST = 10016
RPT = ACC // NS


@functools.partial(
    pl.kernel,
    mesh=plsc.VectorSubcoreMesh(core_axis_name="c", subcore_axis_name="s"),
    compiler_params=pltpu.CompilerParams(use_tc_tiling_on_sc=False,
                                         needs_layout_passes=False),
    out_type=(
        jax.ShapeDtypeStruct((NC, ACC, HD), jnp.float32),
        jax.ShapeDtypeStruct((NS, ACC), jnp.float32),
    ),
    scratch_types=[
        pltpu.VMEM((3, CH), jnp.int32),
        pltpu.VMEM((CH, HD), jnp.float32),
        pltpu.VMEM((CH, HD), jnp.float32),
        pltpu.VMEM((CH, HD), jnp.float32),
        pltpu.VMEM((ACC,), jnp.float32),
        pltpu.VMEM_SHARED((ACC, HD), jnp.float32),
        pltpu.VMEM_SHARED((NR, HD), jnp.float32),
        pltpu.SemaphoreType.DMA,
        pltpu.SemaphoreType.DMA,
    ],
)
def _sc_aggregate(xs_hbm, negrs_hbm, e_hbm,
                  sp_out, degp_out,
                  e_v, xrows, nrows, zbuf, deg_v,
                  acc, negr_sh, sem_x, sem_r):
    cid = lax.axis_index("c")
    sid = lax.axis_index("s")

    z16 = jnp.zeros((16,), jnp.float32)
    ones16 = jnp.ones((16,), jnp.float32)

    def zrow(i, carry):
        for k in range(HD // 16):
            zbuf[i, pl.ds(k * 16, 16)] = z16
        return carry
    lax.fori_loop(0, CH, zrow, 0)

    def zdeg(i, carry):
        deg_v[pl.ds(i * 16, 16)] = z16
        return carry
    lax.fori_loop(0, ACC // 16, zdeg, 0)

    base = sid * RPT
    off = 0
    rem = RPT
    while rem > 0:
        n = min(rem, CH)
        pltpu.sync_copy(zbuf.at[pl.ds(0, n)], acc.at[pl.ds(base + off, n)])
        off += n
        rem -= n

    @pl.when(sid == 0)
    def _():
        pltpu.sync_copy(negrs_hbm.at[cid], negr_sh)

    plsc.subcore_barrier()

    x_half = xs_hbm.at[cid]
    e_tile = e_hbm.at[sid]

    def ebody(j, carry):
        pltpu.sync_copy(e_tile.at[j], e_v)
        pltpu.async_copy(x_half.at[e_v.at[0]], xrows, sem_x).wait()
        pltpu.async_copy(negr_sh.at[e_v.at[2]], nrows, sem_r).wait()
        pltpu.sync_copy(xrows, acc.at[e_v.at[1]], add=True)
        pltpu.sync_copy(nrows, acc.at[e_v.at[1]], add=True)

        @pl.when(cid == 0)
        def _():
            for k in range(CH // 16):
                idx = e_v[1, pl.ds(k * 16, 16)]
                plsc.addupdate_scatter(deg_v, [idx], ones16)
        return carry
    lax.fori_loop(0, CPT, ebody, 0)

    plsc.subcore_barrier()

    pltpu.sync_copy(acc.at[pl.ds(base, RPT)], sp_out.at[cid, pl.ds(base, RPT)])

    @pl.when(cid == 0)
    def _():
        pltpu.sync_copy(deg_v, degp_out.at[sid])


def _ln(h, g, b, eps=1e-5):
    mu = jnp.mean(h, axis=-1, keepdims=True)
    var = jnp.mean((h - mu) ** 2, axis=-1, keepdims=True)
    return (h - mu) / jnp.sqrt(var + eps) * g + b


def _split(y, out_ref):
    out_ref[0] = y[:, :HD]
    out_ref[1] = y[:, HD:]


def _tc_rel_body(rel_ref, W1_ref, g1_ref, b1_ref, W2_ref, g2_ref, b2_ref,
                 r1_ref, n0_ref, n1_ref, r2_ref):
    rel = rel_ref[...]
    _split(-rel, n0_ref)
    h1 = jnp.dot(rel, W1_ref[...], preferred_element_type=jnp.float32)
    r1 = jnp.tanh(_ln(h1, g1_ref[...], b1_ref[...]))
    r1_ref[...] = r1
    _split(-r1, n1_ref)
    h2 = jnp.dot(r1, W2_ref[...], preferred_element_type=jnp.float32)
    r2_ref[...] = jnp.tanh(_ln(h2, g2_ref[...], b2_ref[...]))


def _tc_rel(rel_emb, Wrel1, g_rel1, b_rel1, Wrel2, g_rel2, b_rel2):
    return pl.pallas_call(
        _tc_rel_body,
        out_shape=[
            jax.ShapeDtypeStruct((NR, D), jnp.float32),
            jax.ShapeDtypeStruct((NC, NR, HD), jnp.float32),
            jax.ShapeDtypeStruct((NC, NR, HD), jnp.float32),
            jax.ShapeDtypeStruct((NR, D), jnp.float32),
        ],
    )(rel_emb, Wrel1, g_rel1.reshape(1, D), b_rel1.reshape(1, D),
      Wrel2, g_rel2.reshape(1, D), b_rel2.reshape(1, D))


XB = 1024


def _tc_x_body(sp_ref, deg_ref, time_ref, Wt_ref, bt_ref, gt_ref, btl_ref,
               W_ref, g_ref, b_ref, out_ref, outs_ref):
    t = jnp.dot(time_ref[...], Wt_ref[...],
                preferred_element_type=jnp.float32) + bt_ref[...]
    t = _ln(t, gt_ref[...], btl_ref[...])
    S = jnp.concatenate([sp_ref[0], sp_ref[1]], axis=-1)
    deg = jnp.sum(deg_ref[...], axis=0)
    degc = jnp.maximum(deg, 1.0)
    agg = (S + deg[:, None] * t) / degc[:, None]
    h = jnp.dot(agg, W_ref[...], preferred_element_type=jnp.float32)
    y = jnp.tanh(_ln(h, g_ref[...], b_ref[...]))
    out_ref[...] = y
    _split(y, outs_ref)


def _tc_x(sp, deg, time_emd, W_time, b_time, g_tln, b_tln, W, g, b):
    return pl.pallas_call(
        _tc_x_body,
        grid=(pl.cdiv(NE, XB),),
        in_specs=[
            pl.BlockSpec((NC, XB, HD), lambda i: (0, i, 0)),
            pl.BlockSpec((NS, XB), lambda i: (0, i)),
            pl.BlockSpec((1, D), lambda i: (0, 0)),
            pl.BlockSpec((D, D), lambda i: (0, 0)),
            pl.BlockSpec((1, D), lambda i: (0, 0)),
            pl.BlockSpec((1, D), lambda i: (0, 0)),
            pl.BlockSpec((1, D), lambda i: (0, 0)),
            pl.BlockSpec((D, D), lambda i: (0, 0)),
            pl.BlockSpec((1, D), lambda i: (0, 0)),
            pl.BlockSpec((1, D), lambda i: (0, 0)),
        ],
        out_specs=[
            pl.BlockSpec((XB, D), lambda i: (i, 0)),
            pl.BlockSpec((NC, XB, HD), lambda i: (0, i, 0)),
        ],
        out_shape=[
            jax.ShapeDtypeStruct((NE, D), jnp.float32),
            jax.ShapeDtypeStruct((NC, NE, HD), jnp.float32),
        ],
    )(sp, deg, time_emd.reshape(1, D), W_time, b_time.reshape(1, D),
      g_tln.reshape(1, D), b_tln.reshape(1, D), W, g.reshape(1, D),
      b.reshape(1, D))


def kernel(ent_emb, rel_emb, time_emd, edge_index, edge_type,
           W_time, b_time, g_time_ln, b_time_ln,
           W1, Wrel1, g_ent1, b_ent1, g_rel1, b_rel1,
           W2, Wrel2, g_ent2, b_ent2, g_rel2, b_rel2):
    src = edge_index[0].astype(jnp.int32)
    dst = edge_index[1].astype(jnp.int32)
    et = edge_type.astype(jnp.int32)
    pad = EPAD - E
    srcp = jnp.pad(src, (0, pad)).reshape(NS, CPT, CH)
    dstp = jnp.pad(dst, (0, pad), constant_values=PAD_DST).reshape(NS, CPT, CH)
    etp = jnp.pad(et, (0, pad)).reshape(NS, CPT, CH)
    edges = jnp.stack([srcp, dstp, etp], axis=2)

    r1, negr0s, negr1s, r2 = _tc_rel(rel_emb, Wrel1, g_rel1, b_rel1,
                                     Wrel2, g_rel2, b_rel2)

    xs0 = jnp.stack([ent_emb[:, :HD], ent_emb[:, HD:]])
    sp1, degp = _sc_aggregate(xs0, negr0s, edges)
    x1, xs1 = _tc_x(sp1, degp, time_emd, W_time, b_time, g_time_ln, b_time_ln,
                    W1, g_ent1, b_ent1)
    sp2, _ = _sc_aggregate(xs1, negr1s, edges)
    x2, _ = _tc_x(sp2, degp, time_emd, W_time, b_time, g_time_ln, b_time_ln,
                  W2, g_ent2, b_ent2)
    return (x2, r2)


# speedup vs baseline: 4.1218x; 1.1047x over previous
import functools

import jax
import jax.numpy as jnp
from jax import lax
from jax.experimental import pallas as pl
from jax.experimental.pallas import tpu as pltpu
from jax.experimental.pallas import tpu_sc as plsc

D = 128
HD = D // 2
NE = 10000
NR = 256
E = 320000
NC = 2
NS = 16
CH = 128
CPT = 158
EPT = CH * CPT
EPAD = NS * EPT
ACC = 10112
PAD_DST = 10016
RPT = ACC // NS


@functools.partial(
    pl.kernel,
    mesh=plsc.VectorSubcoreMesh(core_axis_name="c", subcore_axis_name="s"),
    compiler_params=pltpu.CompilerParams(use_tc_tiling_on_sc=False,
                                         needs_layout_passes=False),
    out_type=(
        jax.ShapeDtypeStruct((NC, ACC, HD), jnp.float32),
        jax.ShapeDtypeStruct((NS, ACC), jnp.float32),
    ),
    scratch_types=[
        pltpu.VMEM((3, CH), jnp.int32),
        pltpu.VMEM((CH, HD), jnp.float32),
        pltpu.VMEM((CH, HD), jnp.float32),
        pltpu.VMEM((CH, HD), jnp.float32),
        pltpu.VMEM((ACC,), jnp.float32),
        pltpu.VMEM_SHARED((ACC, HD), jnp.float32),
        pltpu.VMEM_SHARED((NR, HD), jnp.float32),
        pltpu.SemaphoreType.DMA,
        pltpu.SemaphoreType.DMA,
    ],
)
def _sc_aggregate(xs_hbm, negrs_hbm, e_hbm,
                  sp_out, degp_out,
                  e_v, xrows, nrows, zbuf, deg_v,
                  acc, negr_sh, sem_x, sem_r):
    cid = lax.axis_index("c")
    sid = lax.axis_index("s")

    z16 = jnp.zeros((16,), jnp.float32)
    ones16 = jnp.ones((16,), jnp.float32)

    def zrow(i, carry):
        for k in range(HD // 16):
            zbuf[i, pl.ds(k * 16, 16)] = z16
        return carry
    lax.fori_loop(0, CH, zrow, 0)

    def zdeg(i, carry):
        deg_v[pl.ds(i * 16, 16)] = z16
        return carry
    lax.fori_loop(0, ACC // 16, zdeg, 0)

    base = sid * RPT
    off = 0
    rem = RPT
    while rem > 0:
        n = min(rem, CH)
        pltpu.sync_copy(zbuf.at[pl.ds(0, n)], acc.at[pl.ds(base + off, n)])
        off += n
        rem -= n

    @pl.when(sid == 0)
    def _():
        pltpu.sync_copy(negrs_hbm.at[cid], negr_sh)

    plsc.subcore_barrier()

    x_half = xs_hbm.at[cid]
    e_tile = e_hbm.at[sid]

    def ebody(j, carry):
        pltpu.sync_copy(e_tile.at[j], e_v)
        d1 = pltpu.async_copy(x_half.at[e_v.at[0]], xrows, sem_x)
        d2 = pltpu.async_copy(negr_sh.at[e_v.at[2]], nrows, sem_r)
        d1.wait()
        d2.wait()
        pltpu.sync_copy(xrows, acc.at[e_v.at[1]], add=True)
        pltpu.sync_copy(nrows, acc.at[e_v.at[1]], add=True)

        @pl.when(cid == 0)
        def _():
            for k in range(CH // 16):
                idx = e_v[1, pl.ds(k * 16, 16)]
                plsc.addupdate_scatter(deg_v, [idx], ones16)
        return carry
    lax.fori_loop(0, CPT, ebody, 0)

    plsc.subcore_barrier()

    pltpu.sync_copy(acc.at[pl.ds(base, RPT)], sp_out.at[cid, pl.ds(base, RPT)])

    @pl.when(cid == 0)
    def _():
        pltpu.sync_copy(deg_v, degp_out.at[sid])


def _ln(h, g, b, eps=1e-5):
    mu = jnp.mean(h, axis=-1, keepdims=True)
    var = jnp.mean((h - mu) ** 2, axis=-1, keepdims=True)
    return (h - mu) / jnp.sqrt(var + eps) * g + b


def _split(y, out_ref):
    out_ref[0] = y[:, :HD]
    out_ref[1] = y[:, HD:]


def _tc_rel_body(rel_ref, W1_ref, g1_ref, b1_ref, W2_ref, g2_ref, b2_ref,
                 r1_ref, n0_ref, n1_ref, r2_ref):
    rel = rel_ref[...]
    _split(-rel, n0_ref)
    h1 = jnp.dot(rel, W1_ref[...], preferred_element_type=jnp.float32)
    r1 = jnp.tanh(_ln(h1, g1_ref[...], b1_ref[...]))
    r1_ref[...] = r1
    _split(-r1, n1_ref)
    h2 = jnp.dot(r1, W2_ref[...], preferred_element_type=jnp.float32)
    r2_ref[...] = jnp.tanh(_ln(h2, g2_ref[...], b2_ref[...]))


def _tc_rel(rel_emb, Wrel1, g_rel1, b_rel1, Wrel2, g_rel2, b_rel2):
    return pl.pallas_call(
        _tc_rel_body,
        out_shape=[
            jax.ShapeDtypeStruct((NR, D), jnp.float32),
            jax.ShapeDtypeStruct((NC, NR, HD), jnp.float32),
            jax.ShapeDtypeStruct((NC, NR, HD), jnp.float32),
            jax.ShapeDtypeStruct((NR, D), jnp.float32),
        ],
    )(rel_emb, Wrel1, g_rel1.reshape(1, D), b_rel1.reshape(1, D),
      Wrel2, g_rel2.reshape(1, D), b_rel2.reshape(1, D))


XB = 1024


def _tc_x_body(sp_ref, deg_ref, time_ref, Wt_ref, bt_ref, gt_ref, btl_ref,
               W_ref, g_ref, b_ref, out_ref, outs_ref):
    t = jnp.dot(time_ref[...], Wt_ref[...],
                preferred_element_type=jnp.float32) + bt_ref[...]
    t = _ln(t, gt_ref[...], btl_ref[...])
    S = jnp.concatenate([sp_ref[0], sp_ref[1]], axis=-1)
    deg = jnp.sum(deg_ref[...], axis=0)
    degc = jnp.maximum(deg, 1.0)
    agg = (S + deg[:, None] * t) / degc[:, None]
    h = jnp.dot(agg, W_ref[...], preferred_element_type=jnp.float32)
    y = jnp.tanh(_ln(h, g_ref[...], b_ref[...]))
    out_ref[...] = y
    _split(y, outs_ref)


def _tc_x(sp, deg, time_emd, W_time, b_time, g_tln, b_tln, W, g, b):
    return pl.pallas_call(
        _tc_x_body,
        grid=(pl.cdiv(NE, XB),),
        in_specs=[
            pl.BlockSpec((NC, XB, HD), lambda i: (0, i, 0)),
            pl.BlockSpec((NS, XB), lambda i: (0, i)),
            pl.BlockSpec((1, D), lambda i: (0, 0)),
            pl.BlockSpec((D, D), lambda i: (0, 0)),
            pl.BlockSpec((1, D), lambda i: (0, 0)),
            pl.BlockSpec((1, D), lambda i: (0, 0)),
            pl.BlockSpec((1, D), lambda i: (0, 0)),
            pl.BlockSpec((D, D), lambda i: (0, 0)),
            pl.BlockSpec((1, D), lambda i: (0, 0)),
            pl.BlockSpec((1, D), lambda i: (0, 0)),
        ],
        out_specs=[
            pl.BlockSpec((XB, D), lambda i: (i, 0)),
            pl.BlockSpec((NC, XB, HD), lambda i: (0, i, 0)),
        ],
        out_shape=[
            jax.ShapeDtypeStruct((NE, D), jnp.float32),
            jax.ShapeDtypeStruct((NC, NE, HD), jnp.float32),
        ],
    )(sp, deg, time_emd.reshape(1, D), W_time, b_time.reshape(1, D),
      g_tln.reshape(1, D), b_tln.reshape(1, D), W, g.reshape(1, D),
      b.reshape(1, D))


def kernel(ent_emb, rel_emb, time_emd, edge_index, edge_type,
           W_time, b_time, g_time_ln, b_time_ln,
           W1, Wrel1, g_ent1, b_ent1, g_rel1, b_rel1,
           W2, Wrel2, g_ent2, b_ent2, g_rel2, b_rel2):
    src = edge_index[0].astype(jnp.int32)
    dst = edge_index[1].astype(jnp.int32)
    et = edge_type.astype(jnp.int32)
    pad = EPAD - E
    srcp = jnp.pad(src, (0, pad)).reshape(NS, CPT, CH)
    dstp = jnp.pad(dst, (0, pad), constant_values=PAD_DST).reshape(NS, CPT, CH)
    etp = jnp.pad(et, (0, pad)).reshape(NS, CPT, CH)
    edges = jnp.stack([srcp, dstp, etp], axis=2)

    r1, negr0s, negr1s, r2 = _tc_rel(rel_emb, Wrel1, g_rel1, b_rel1,
                                     Wrel2, g_rel2, b_rel2)

    xs0 = jnp.stack([ent_emb[:, :HD], ent_emb[:, HD:]])
    sp1, degp = _sc_aggregate(xs0, negr0s, edges)
    x1, xs1 = _tc_x(sp1, degp, time_emd, W_time, b_time, g_time_ln, b_time_ln,
                    W1, g_ent1, b_ent1)
    sp2, _ = _sc_aggregate(xs1, negr1s, edges)
    x2, _ = _tc_x(sp2, degp, time_emd, W_time, b_time, g_time_ln, b_time_ln,
                  W2, g_ent2, b_ent2)
    return (x2, r2)
